# baseline (speedup 1.0000x reference)
import functools

import jax
import jax.numpy as jnp
from jax import lax
from jax.experimental import pallas as pl
from jax.experimental.pallas import tpu as pltpu
from jax.experimental.pallas import tpu_sc as plsc

N_NODES = 10000
N_PAD = 10240
E_EDGES = 320000
E_PAD = 327680
E_HALF = E_PAD // 2
EDGE_TILE = 4096
NODE_TILE = 2048


def _pad_rows(a, n):
    return jnp.pad(a, ((0, n - a.shape[0]),) + ((0, 0),) * (a.ndim - 1))


def _zero_stuff(W, pieces, total):
    out = jnp.zeros((total, W.shape[1]), W.dtype)
    for dst_off, src_off, ln in pieces:
        out = out.at[dst_off:dst_off + ln].set(W[src_off:src_off + ln])
    return out


def _prep_layers(layers, first_pieces=None, first_total=None, out_pad=None):
    prepped = []
    nl = len(layers)
    for i, (W, b) in enumerate(layers):
        if i == 0 and first_pieces is not None:
            W = _zero_stuff(W, first_pieces, first_total)
        if i == nl - 1 and out_pad is not None and W.shape[1] < out_pad:
            W = jnp.pad(W, ((0, 0), (0, out_pad - W.shape[1])))
            b = jnp.pad(b, (0, out_pad - b.shape[0]))
        prepped.append((W, b.reshape(1, -1)))
    return prepped


def _tc_mlp(inputs, heads, *, tile):
    R = inputs[0].shape[0]
    grid = (R // tile,)
    n_in = len(inputs)
    in_specs = [pl.BlockSpec((tile, a.shape[1]), lambda i: (i, 0)) for a in inputs]
    ops = list(inputs)
    for h in heads:
        for (W, b) in h["layers"]:
            ops.append(W)
            in_specs.append(pl.BlockSpec(W.shape, lambda i: (0, 0)))
            ops.append(b)
            in_specs.append(pl.BlockSpec(b.shape, lambda i: (0, 0)))
        if h.get("alpha") is not None:
            ops.append(jnp.reshape(h["alpha"], (1, 1)).astype(jnp.float32))
            in_specs.append(pl.BlockSpec((1, 1), lambda i: (0, 0)))
    out_shapes = [
        jax.ShapeDtypeStruct((R, h["layers"][-1][0].shape[1]), jnp.float32)
        for h in heads
    ]
    out_specs = [
        pl.BlockSpec((tile, s.shape[1]), lambda i: (i, 0)) for s in out_shapes
    ]

    def body(*refs):
        ins = refs[:n_in]
        outs = refs[len(refs) - len(heads):]
        if n_in > 1:
            X = jnp.concatenate([r[...] for r in ins], axis=1)
        else:
            X = ins[0][...]
        pos = n_in
        for hi, h in enumerate(heads):
            H = X
            nl = len(h["layers"])
            for li in range(nl):
                W = refs[pos][...]
                b = refs[pos + 1][...]
                pos += 2
                H = jnp.dot(H, W, preferred_element_type=jnp.float32) + b
                if li < nl - 1:
                    H = jnp.maximum(H, 0.0)
            oa = h.get("out_act")
            if oa == "relu":
                H = jnp.maximum(H, 0.0)
            elif oa == "sigmoid":
                H = jax.nn.sigmoid(H)
            if h.get("alpha") is not None:
                a = refs[pos][...]
                pos += 1
                H = a * ins[0][...] + (1.0 - a) * H
            outs[hi][...] = H

    res = pl.pallas_call(
        body,
        grid=grid,
        in_specs=in_specs,
        out_specs=out_specs,
        out_shape=out_shapes,
    )(*ops)
    return list(res)


@functools.cache
def _sc_mesh():
    return plsc.VectorSubcoreMesh(core_axis_name="c", subcore_axis_name="s")


_NW = 32
_GCH = 2 * E_HALF // 128
_GPW = _GCH // _NW
_FCH = E_HALF * 4 // 128
_FPW = _FCH // _NW
_FSUB = N_PAD * 4 // 16


def _sc_gather(h8, ii3):
    @functools.partial(
        pl.kernel,
        out_type=jax.ShapeDtypeStruct((_GCH, 128, 8), jnp.float32),
        mesh=_sc_mesh(),
        scratch_types=[
            pltpu.VMEM((_GPW, 128), jnp.int32),
            pltpu.VMEM((_GPW, 128, 8), jnp.float32),
            pltpu.SemaphoreType.DMA,
        ],
        compiler_params=pltpu.CompilerParams(use_tc_tiling_on_sc=False),
    )
    def gk(h_hbm, ii_hbm, out_hbm, idx_v, rows_v, sem):
        w = lax.axis_index("s") * 2 + lax.axis_index("c")
        base = w * _GPW
        pltpu.sync_copy(ii_hbm.at[pl.ds(base, _GPW)], idx_v)
        for j0 in range(0, _GPW, 16):
            cps = [
                pltpu.async_copy(
                    h_hbm.at[idx_v.at[j0 + j]], rows_v.at[j0 + j], sem)
                for j in range(16)
            ]
            for cp in cps:
                cp.wait()
        pltpu.sync_copy(rows_v, out_hbm.at[pl.ds(base, _GPW)])

    out = gk(h8, ii3)
    return out.reshape(E_HALF, 16)


def _sc_scatter(m, dst4):
    mf = m.reshape(_FCH, 128)
    zeros = jnp.zeros((N_PAD * 4,), jnp.float32)

    @functools.partial(
        pl.kernel,
        out_type=jax.ShapeDtypeStruct((2, N_PAD * 4), jnp.float32),
        mesh=_sc_mesh(),
        scratch_types=[
            pltpu.VMEM((_FPW, 128), jnp.int32),
            pltpu.VMEM((_FPW, 128), jnp.float32),
            pltpu.VMEM_SHARED((N_PAD * 4,), jnp.float32),
            pltpu.SemaphoreType.DMA,
        ],
        compiler_params=pltpu.CompilerParams(use_tc_tiling_on_sc=False),
    )
    def sk(m_hbm, dst_hbm, z_hbm, out_hbm, idx_v, m_v, agg_sh, sem):
        c = lax.axis_index("c")
        s = lax.axis_index("s")
        w = s * 2 + c
        pltpu.sync_copy(z_hbm.at[pl.ds(s * _FSUB, _FSUB)],
                        agg_sh.at[pl.ds(s * _FSUB, _FSUB)])
        plsc.subcore_barrier()
        base = w * _FPW
        pltpu.sync_copy(dst_hbm.at[pl.ds(base, _FPW)], idx_v)
        pltpu.sync_copy(m_hbm.at[pl.ds(base, _FPW)], m_v)
        for j0 in range(0, _FPW, 16):
            cps = [
                pltpu.async_copy(m_v.at[j0 + j], agg_sh.at[idx_v.at[j0 + j]],
                                 sem, add=True)
                for j in range(16)
            ]
            for cp in cps:
                cp.wait()
        plsc.subcore_barrier()
        pltpu.sync_copy(agg_sh.at[pl.ds(s * _FSUB, _FSUB)],
                        out_hbm.at[c].at[pl.ds(s * _FSUB, _FSUB)])

    out = sk(mf, dst4, zeros)
    return out.reshape(2, N_PAD, 4)


def _edge_mlp_half(lp, hpair, e):
    W1, b1 = lp["edge"][0]
    pieces = [(0, 0, 5), (8, 5, 5), (16, 10, 4)]
    elayers = _prep_layers(
        [(W1, b1)] + lp["edge"][1:], first_pieces=pieces, first_total=20
    )
    (m,) = _tc_mlp(
        [hpair, e], [{"layers": elayers, "out_act": None}], tile=EDGE_TILE
    )
    return m


def _in_layer(lp, h8, iiA, iiB, eA, eB, dstA, dstB, alpha):
    gA = _sc_gather(h8, iiA)
    gB = _sc_gather(h8, iiB)
    mA = _edge_mlp_half(lp, gA, eA)
    aggA = _sc_scatter(mA, dstA)
    mB = _edge_mlp_half(lp, gB, eB)
    aggB = _sc_scatter(mB, dstB)
    agg = aggA[0] + aggA[1] + aggB[0] + aggB[1]
    Wn, bn = lp["node"][0]
    npieces = [(0, 0, 5), (8, 5, 4)]
    nlayers = _prep_layers(
        [(Wn, bn)] + lp["node"][1:], first_pieces=npieces, first_total=12,
        out_pad=8,
    )
    (h8_new,) = _tc_mlp(
        [h8, agg],
        [{"layers": nlayers, "out_act": None, "alpha": alpha}],
        tile=NODE_TILE,
    )
    return h8_new, mA, mB


def kernel(x, edge_index, edge_attr, params, alpha_ec=0.5, alpha_hc=0.5):
    src = edge_index[0]
    dst = edge_index[1]
    pad_idx = jnp.full((E_PAD - E_EDGES,), N_NODES, dtype=jnp.int32)
    src_p = jnp.concatenate([src, pad_idx])
    dst_p = jnp.concatenate([dst, pad_idx])
    ii3 = jnp.stack([dst_p, src_p], axis=1).reshape(2 * _GCH, 128)
    iiA, iiB = ii3[:_GCH], ii3[_GCH:]
    dst4 = (dst_p[:, None] * 4
            + jnp.arange(4, dtype=jnp.int32)[None, :]).reshape(2 * _FCH, 128)
    dstA, dstB = dst4[:_FCH], dst4[_FCH:]
    ea_p = _pad_rows(edge_attr, E_PAD)
    eaA, eaB = ea_p[:E_HALF], ea_p[E_HALF:]
    x_p = _pad_rows(x, N_PAD)

    enc_heads = []
    for name in ("ec_node_enc", "hc_node_enc"):
        enc_heads.append({
            "layers": _prep_layers(params[name], out_pad=8),
            "out_act": "relu",
        })
    h_ec8, h_hc8 = _tc_mlp([x_p], enc_heads, tile=NODE_TILE)

    enc_e = {"layers": _prep_layers(params["ec_edge_enc"]), "out_act": "relu"}
    (eA,) = _tc_mlp([eaA], [enc_e], tile=EDGE_TILE)
    (eB,) = _tc_mlp([eaB], [enc_e], tile=EDGE_TILE)

    e_listA, e_listB = [eA], [eB]
    for lp in params["ec_layers"]:
        h_ec8, eA, eB = _in_layer(
            lp, h_ec8, iiA, iiB, eA, eB, dstA, dstB, alpha_ec)
        e_listA.append(eA)
        e_listB.append(eB)

    w_head = {"layers": _prep_layers(params["W"]), "out_act": "sigmoid"}
    (ewA,) = _tc_mlp(e_listA, [w_head], tile=EDGE_TILE)
    (ewB,) = _tc_mlp(e_listB, [w_head], tile=EDGE_TILE)

    hc_enc = {"layers": _prep_layers(params["hc_edge_enc"]), "out_act": "relu"}
    (eA,) = _tc_mlp([ewA, eaA], [hc_enc], tile=EDGE_TILE)
    (eB,) = _tc_mlp([ewB, eaB], [hc_enc], tile=EDGE_TILE)

    eh_listA, eh_listB = [eA], [eB]
    for lp in params["hc_layers"]:
        h_hc8, eA, eB = _in_layer(
            lp, h_hc8, iiA, iiB, eA, eB, dstA, dstB, alpha_hc)
        eh_listA.append(eA)
        eh_listB.append(eB)

    h5 = [(0, 0, 5)]
    beta_p, hout_p = _tc_mlp(
        [h_hc8],
        [
            {"layers": _prep_layers(params["B"], first_pieces=h5,
                                    first_total=8), "out_act": "sigmoid"},
            {"layers": _prep_layers(params["X"], first_pieces=h5,
                                    first_total=8), "out_act": None},
        ],
        tile=NODE_TILE,
    )

    Wp1, bp1 = params["P"]["edge"][0]
    ppieces = [(0, 0, 5), (8, 5, 5), (16, 10, 16)]
    playersE = _prep_layers(
        [(Wp1, bp1)] + params["P"]["edge"][1:],
        first_pieces=ppieces, first_total=32, out_pad=4,
    )
    p_head = {"layers": playersE, "out_act": None}
    gA = _sc_gather(h_hc8, iiA)
    gB = _sc_gather(h_hc8, iiB)
    (mpA,) = _tc_mlp([gA] + eh_listA, [p_head], tile=EDGE_TILE)
    aggpA = _sc_scatter(mpA, dstA)
    (mpB,) = _tc_mlp([gB] + eh_listB, [p_head], tile=EDGE_TILE)
    aggpB = _sc_scatter(mpB, dstB)
    aggp = aggpA[0] + aggpA[1] + aggpB[0] + aggpB[1]
    Wpn, bpn = params["P"]["node"][0]
    npieces = [(0, 0, 5), (8, 5, 1)]
    playersN = _prep_layers(
        [(Wpn, bpn)] + params["P"]["node"][1:],
        first_pieces=npieces, first_total=12,
    )
    (track_p,) = _tc_mlp(
        [h_hc8, aggp],
        [{"layers": playersN, "out_act": None}],
        tile=NODE_TILE,
    )

    edge_weights = jnp.concatenate([ewA, ewB])[:E_EDGES]
    h = hout_p[:N_NODES]
    beta = beta_p[:N_NODES]
    track_params = track_p[:N_NODES]
    return (edge_weights, h, beta, track_params)


# speedup vs baseline: 1.0310x; 1.0310x over previous
import functools

import jax
import jax.numpy as jnp
from jax import lax
from jax.experimental import pallas as pl
from jax.experimental.pallas import tpu as pltpu
from jax.experimental.pallas import tpu_sc as plsc

N_NODES = 10000
N_PAD = 10240
E_EDGES = 320000
E_PAD = 327680
EDGE_TILE = 4096
NODE_TILE = 2048


def _pad_rows(a, n):
    return jnp.pad(a, ((0, n - a.shape[0]),) + ((0, 0),) * (a.ndim - 1))


def _zero_stuff(W, pieces, total):
    out = jnp.zeros((total, W.shape[1]), W.dtype)
    for dst_off, src_off, ln in pieces:
        out = out.at[dst_off:dst_off + ln].set(W[src_off:src_off + ln])
    return out


def _prep_layers(layers, first_pieces=None, first_total=None, out_pad=None):
    prepped = []
    nl = len(layers)
    for i, (W, b) in enumerate(layers):
        if i == 0 and first_pieces is not None:
            W = _zero_stuff(W, first_pieces, first_total)
        if i == nl - 1 and out_pad is not None and W.shape[1] < out_pad:
            W = jnp.pad(W, ((0, 0), (0, out_pad - W.shape[1])))
            b = jnp.pad(b, (0, out_pad - b.shape[0]))
        prepped.append((W, b.reshape(1, -1)))
    return prepped


def _tc_mlp(inputs, heads, *, tile):
    R = inputs[0].shape[0]
    grid = (R // tile,)
    n_in = len(inputs)
    in_specs = [pl.BlockSpec((tile, a.shape[1]), lambda i: (i, 0)) for a in inputs]
    ops = list(inputs)
    for h in heads:
        for (W, b) in h["layers"]:
            ops.append(W)
            in_specs.append(pl.BlockSpec(W.shape, lambda i: (0, 0)))
            ops.append(b)
            in_specs.append(pl.BlockSpec(b.shape, lambda i: (0, 0)))
        if h.get("alpha") is not None:
            ops.append(jnp.reshape(h["alpha"], (1, 1)).astype(jnp.float32))
            in_specs.append(pl.BlockSpec((1, 1), lambda i: (0, 0)))
    out_shapes = [
        jax.ShapeDtypeStruct((R, h["layers"][-1][0].shape[1]), jnp.float32)
        for h in heads
    ]
    out_specs = [
        pl.BlockSpec((tile, s.shape[1]), lambda i: (i, 0)) for s in out_shapes
    ]

    def body(*refs):
        ins = refs[:n_in]
        outs = refs[len(refs) - len(heads):]
        if n_in > 1:
            X = jnp.concatenate([r[...] for r in ins], axis=1)
        else:
            X = ins[0][...]
        pos = n_in
        for hi, h in enumerate(heads):
            H = X
            nl = len(h["layers"])
            for li in range(nl):
                W = refs[pos][...]
                b = refs[pos + 1][...]
                pos += 2
                H = jnp.dot(H, W, preferred_element_type=jnp.float32) + b
                if li < nl - 1:
                    H = jnp.maximum(H, 0.0)
            oa = h.get("out_act")
            if oa == "relu":
                H = jnp.maximum(H, 0.0)
            elif oa == "sigmoid":
                H = jax.nn.sigmoid(H)
            if h.get("alpha") is not None:
                a = refs[pos][...]
                pos += 1
                H = a * ins[0][...] + (1.0 - a) * H
            outs[hi][...] = H

    res = pl.pallas_call(
        body,
        grid=grid,
        in_specs=in_specs,
        out_specs=out_specs,
        out_shape=out_shapes,
    )(*ops)
    return list(res)


@functools.cache
def _sc_mesh():
    return plsc.VectorSubcoreMesh(core_axis_name="c", subcore_axis_name="s")


_NW = 32
_GCHUNKS = 2 * E_PAD // 128
_GPW = _GCHUNKS // _NW
_GGRP = _GPW // 2
_FCHUNKS = E_PAD * 4 // 128
_FPW = _FCHUNKS // _NW
_FSUB = N_PAD * 4 // 16


def _sc_gather(h8, ii3):
    @functools.partial(
        pl.kernel,
        out_type=jax.ShapeDtypeStruct((_GCHUNKS, 128, 8), jnp.float32),
        mesh=_sc_mesh(),
        scratch_types=[
            pltpu.VMEM((_GGRP, 128), jnp.int32),
            pltpu.VMEM((_GGRP, 128, 8), jnp.float32),
            pltpu.SemaphoreType.DMA,
        ],
        compiler_params=pltpu.CompilerParams(use_tc_tiling_on_sc=False),
    )
    def gk(h_hbm, ii_hbm, out_hbm, idx_v, rows_v, sem):
        w = lax.axis_index("s") * 2 + lax.axis_index("c")
        for g in range(2):
            base = w * _GPW + g * _GGRP
            pltpu.sync_copy(ii_hbm.at[pl.ds(base, _GGRP)], idx_v)
            for j0 in range(0, _GGRP, 16):
                for j in range(16):
                    pltpu.async_copy(
                        h_hbm.at[idx_v.at[j0 + j]], rows_v.at[j0 + j], sem)
                pltpu.make_async_copy(
                    out_hbm.at[pl.ds(0, 16)],
                    rows_v.at[pl.ds(j0, 16)], sem).wait()
            pltpu.sync_copy(rows_v, out_hbm.at[pl.ds(base, _GGRP)])

    out = gk(h8, ii3)
    return out.reshape(E_PAD, 16)


def _sc_scatter(m, dst4):
    mf = m.reshape(_FCHUNKS, 128)
    zeros = jnp.zeros((N_PAD * 4,), jnp.float32)

    @functools.partial(
        pl.kernel,
        out_type=jax.ShapeDtypeStruct((2, N_PAD * 4), jnp.float32),
        mesh=_sc_mesh(),
        scratch_types=[
            pltpu.VMEM((_FPW, 128), jnp.int32),
            pltpu.VMEM((_FPW, 128), jnp.float32),
            pltpu.VMEM_SHARED((N_PAD * 4,), jnp.float32),
            pltpu.SemaphoreType.DMA,
        ],
        compiler_params=pltpu.CompilerParams(use_tc_tiling_on_sc=False),
    )
    def sk(m_hbm, dst_hbm, z_hbm, out_hbm, idx_v, m_v, agg_sh, sem):
        c = lax.axis_index("c")
        s = lax.axis_index("s")
        w = s * 2 + c
        pltpu.sync_copy(z_hbm.at[pl.ds(s * _FSUB, _FSUB)],
                        agg_sh.at[pl.ds(s * _FSUB, _FSUB)])
        plsc.subcore_barrier()
        base = w * _FPW
        pltpu.sync_copy(dst_hbm.at[pl.ds(base, _FPW)], idx_v)
        pltpu.sync_copy(m_hbm.at[pl.ds(base, _FPW)], m_v)
        for j0 in range(0, _FPW, 16):
            for j in range(16):
                pltpu.async_copy(m_v.at[j0 + j], agg_sh.at[idx_v.at[j0 + j]],
                                 sem, add=True)
            pltpu.make_async_copy(
                m_hbm.at[pl.ds(0, 16)],
                m_v.at[pl.ds(j0, 16)], sem).wait()
        plsc.subcore_barrier()
        pltpu.sync_copy(agg_sh.at[pl.ds(s * _FSUB, _FSUB)],
                        out_hbm.at[c].at[pl.ds(s * _FSUB, _FSUB)])

    out = sk(mf, dst4, zeros)
    return out.reshape(2, N_PAD, 4)


def _in_layer(lp, h8, hpair, e, dst4, alpha):
    W1, b1 = lp["edge"][0]
    pieces = [(0, 0, 5), (8, 5, 5), (16, 10, 4)]
    elayers = _prep_layers(
        [(W1, b1)] + lp["edge"][1:], first_pieces=pieces, first_total=20
    )
    (m,) = _tc_mlp(
        [hpair, e], [{"layers": elayers, "out_act": None}], tile=EDGE_TILE
    )
    agg2 = _sc_scatter(m, dst4)
    Wn, bn = lp["node"][0]
    npieces = [(0, 0, 5), (8, 5, 4)]
    nlayers = _prep_layers(
        [(Wn, bn)] + lp["node"][1:], first_pieces=npieces, first_total=12,
        out_pad=8,
    )
    agg = agg2[0] + agg2[1]
    (h8_new,) = _tc_mlp(
        [h8, agg],
        [{"layers": nlayers, "out_act": None, "alpha": alpha}],
        tile=NODE_TILE,
    )
    return h8_new, m


def kernel(x, edge_index, edge_attr, params, alpha_ec=0.5, alpha_hc=0.5):
    src = edge_index[0]
    dst = edge_index[1]
    pad_idx = jnp.full((E_PAD - E_EDGES,), N_NODES, dtype=jnp.int32)
    src_p = jnp.concatenate([src, pad_idx])
    dst_p = jnp.concatenate([dst, pad_idx])
    ii3 = jnp.stack([dst_p, src_p], axis=1).reshape(_GCHUNKS, 128)
    dst4 = (dst_p[:, None] * 4
            + jnp.arange(4, dtype=jnp.int32)[None, :]).reshape(_FCHUNKS, 128)
    ea_p = _pad_rows(edge_attr, E_PAD)
    x_p = _pad_rows(x, N_PAD)

    enc_heads = []
    for name in ("ec_node_enc", "hc_node_enc"):
        enc_heads.append({
            "layers": _prep_layers(params[name], out_pad=8),
            "out_act": "relu",
        })
    h_ec8, h_hc8 = _tc_mlp([x_p], enc_heads, tile=NODE_TILE)

    (e_ec,) = _tc_mlp(
        [ea_p],
        [{"layers": _prep_layers(params["ec_edge_enc"]), "out_act": "relu"}],
        tile=EDGE_TILE,
    )

    e_list = [e_ec]
    e_cur = e_ec
    for lp in params["ec_layers"]:
        hpair = _sc_gather(h_ec8, ii3)
        h_ec8, m = _in_layer(lp, h_ec8, hpair, e_cur, dst4, alpha_ec)
        e_list.append(m)
        e_cur = m

    (ew_p,) = _tc_mlp(
        e_list,
        [{"layers": _prep_layers(params["W"]), "out_act": "sigmoid"}],
        tile=EDGE_TILE,
    )

    (e_hc,) = _tc_mlp(
        [ew_p, ea_p],
        [{"layers": _prep_layers(params["hc_edge_enc"]), "out_act": "relu"}],
        tile=EDGE_TILE,
    )

    eh_list = [e_hc]
    e_cur = e_hc
    for lp in params["hc_layers"]:
        hpair = _sc_gather(h_hc8, ii3)
        h_hc8, m = _in_layer(lp, h_hc8, hpair, e_cur, dst4, alpha_hc)
        eh_list.append(m)
        e_cur = m

    h5 = [(0, 0, 5)]
    beta_p, hout_p = _tc_mlp(
        [h_hc8],
        [
            {"layers": _prep_layers(params["B"], first_pieces=h5,
                                    first_total=8), "out_act": "sigmoid"},
            {"layers": _prep_layers(params["X"], first_pieces=h5,
                                    first_total=8), "out_act": None},
        ],
        tile=NODE_TILE,
    )

    Wp1, bp1 = params["P"]["edge"][0]
    ppieces = [(0, 0, 5), (8, 5, 5), (16, 10, 16)]
    playersE = _prep_layers(
        [(Wp1, bp1)] + params["P"]["edge"][1:],
        first_pieces=ppieces, first_total=32, out_pad=4,
    )
    hpair = _sc_gather(h_hc8, ii3)
    (mp,) = _tc_mlp(
        [hpair] + eh_list,
        [{"layers": playersE, "out_act": None}],
        tile=EDGE_TILE,
    )
    aggp2 = _sc_scatter(mp, dst4)
    aggp = aggp2[0] + aggp2[1]
    Wpn, bpn = params["P"]["node"][0]
    npieces = [(0, 0, 5), (8, 5, 1)]
    playersN = _prep_layers(
        [(Wpn, bpn)] + params["P"]["node"][1:],
        first_pieces=npieces, first_total=12,
    )
    (track_p,) = _tc_mlp(
        [h_hc8, aggp],
        [{"layers": playersN, "out_act": None}],
        tile=NODE_TILE,
    )

    edge_weights = ew_p[:E_EDGES]
    h = hout_p[:N_NODES]
    beta = beta_p[:N_NODES]
    track_params = track_p[:N_NODES]
    return (edge_weights, h, beta, track_params)


# speedup vs baseline: 1.0518x; 1.0202x over previous
import functools

import jax
import jax.numpy as jnp
from jax import lax
from jax.experimental import pallas as pl
from jax.experimental.pallas import tpu as pltpu
from jax.experimental.pallas import tpu_sc as plsc

N_NODES = 10000
N_PAD = 10240
E_EDGES = 320000
E_PAD = 327680
EDGE_TILE = 8192
NODE_TILE = 2048


def _pad_rows(a, n):
    return jnp.pad(a, ((0, n - a.shape[0]),) + ((0, 0),) * (a.ndim - 1))


def _zero_stuff(W, pieces, total):
    out = jnp.zeros((total, W.shape[1]), W.dtype)
    for dst_off, src_off, ln in pieces:
        out = out.at[dst_off:dst_off + ln].set(W[src_off:src_off + ln])
    return out


def _prep_layers(layers, first_pieces=None, first_total=None, out_pad=None):
    prepped = []
    nl = len(layers)
    for i, (W, b) in enumerate(layers):
        if i == 0 and first_pieces is not None:
            W = _zero_stuff(W, first_pieces, first_total)
        if i == nl - 1 and out_pad is not None and W.shape[1] < out_pad:
            W = jnp.pad(W, ((0, 0), (0, out_pad - W.shape[1])))
            b = jnp.pad(b, (0, out_pad - b.shape[0]))
        prepped.append((W, b.reshape(1, -1)))
    return prepped


def _tc_mlp(inputs, heads, *, tile):
    R = inputs[0].shape[0]
    grid = (R // tile,)
    n_in = len(inputs)
    in_specs = [pl.BlockSpec((tile, a.shape[1]), lambda i: (i, 0)) for a in inputs]
    ops = list(inputs)
    for h in heads:
        for (W, b) in h["layers"]:
            ops.append(W)
            in_specs.append(pl.BlockSpec(W.shape, lambda i: (0, 0)))
            ops.append(b)
            in_specs.append(pl.BlockSpec(b.shape, lambda i: (0, 0)))
        if h.get("alpha") is not None:
            ops.append(jnp.reshape(h["alpha"], (1, 1)).astype(jnp.float32))
            in_specs.append(pl.BlockSpec((1, 1), lambda i: (0, 0)))
    out_shapes = [
        jax.ShapeDtypeStruct((R, h["layers"][-1][0].shape[1]), jnp.float32)
        for h in heads
    ]
    out_specs = [
        pl.BlockSpec((tile, s.shape[1]), lambda i: (i, 0)) for s in out_shapes
    ]

    def body(*refs):
        ins = refs[:n_in]
        outs = refs[len(refs) - len(heads):]
        if n_in > 1:
            X = jnp.concatenate([r[...] for r in ins], axis=1)
        else:
            X = ins[0][...]
        pos = n_in
        for hi, h in enumerate(heads):
            H = X
            nl = len(h["layers"])
            for li in range(nl):
                W = refs[pos][...]
                b = refs[pos + 1][...]
                pos += 2
                H = jnp.dot(H, W, preferred_element_type=jnp.float32) + b
                if li < nl - 1:
                    H = jnp.maximum(H, 0.0)
            oa = h.get("out_act")
            if oa == "relu":
                H = jnp.maximum(H, 0.0)
            elif oa == "sigmoid":
                H = jax.nn.sigmoid(H)
            if h.get("alpha") is not None:
                a = refs[pos][...]
                pos += 1
                H = a * ins[0][...] + (1.0 - a) * H
            outs[hi][...] = H

    res = pl.pallas_call(
        body,
        grid=grid,
        in_specs=in_specs,
        out_specs=out_specs,
        out_shape=out_shapes,
    )(*ops)
    return list(res)


@functools.cache
def _sc_mesh():
    return plsc.VectorSubcoreMesh(core_axis_name="c", subcore_axis_name="s")


_NW = 32
_GCHUNKS = 2 * E_PAD // 128
_GPW = _GCHUNKS // _NW
_GGRP = _GPW // 2
_FCHUNKS = E_PAD * 4 // 128
_FPW = _FCHUNKS // _NW
_FSUB = N_PAD * 4 // 16


def _sc_gather(h8, ii3):
    @functools.partial(
        pl.kernel,
        out_type=jax.ShapeDtypeStruct((_GCHUNKS, 128, 8), jnp.float32),
        mesh=_sc_mesh(),
        scratch_types=[
            pltpu.VMEM((_GGRP, 128), jnp.int32),
            pltpu.VMEM((_GGRP, 128, 8), jnp.float32),
            pltpu.SemaphoreType.DMA,
        ],
        compiler_params=pltpu.CompilerParams(use_tc_tiling_on_sc=False),
    )
    def gk(h_hbm, ii_hbm, out_hbm, idx_v, rows_v, sem):
        w = lax.axis_index("s") * 2 + lax.axis_index("c")
        for g in range(2):
            base = w * _GPW + g * _GGRP
            pltpu.sync_copy(ii_hbm.at[pl.ds(base, _GGRP)], idx_v)
            for j0 in range(0, _GGRP, 16):
                for j in range(16):
                    pltpu.async_copy(
                        h_hbm.at[idx_v.at[j0 + j]], rows_v.at[j0 + j], sem)
                pltpu.make_async_copy(
                    out_hbm.at[pl.ds(0, 16)],
                    rows_v.at[pl.ds(j0, 16)], sem).wait()
            pltpu.sync_copy(rows_v, out_hbm.at[pl.ds(base, _GGRP)])

    out = gk(h8, ii3)
    return out.reshape(E_PAD, 16)


def _sc_scatter(m, dst4):
    mf = m.reshape(_FCHUNKS, 128)
    zeros = jnp.zeros((N_PAD * 4,), jnp.float32)

    @functools.partial(
        pl.kernel,
        out_type=jax.ShapeDtypeStruct((2, N_PAD * 4), jnp.float32),
        mesh=_sc_mesh(),
        scratch_types=[
            pltpu.VMEM((_FPW, 128), jnp.int32),
            pltpu.VMEM((_FPW, 128), jnp.float32),
            pltpu.VMEM_SHARED((N_PAD * 4,), jnp.float32),
            pltpu.SemaphoreType.DMA,
        ],
        compiler_params=pltpu.CompilerParams(use_tc_tiling_on_sc=False),
    )
    def sk(m_hbm, dst_hbm, z_hbm, out_hbm, idx_v, m_v, agg_sh, sem):
        c = lax.axis_index("c")
        s = lax.axis_index("s")
        w = s * 2 + c
        pltpu.sync_copy(z_hbm.at[pl.ds(s * _FSUB, _FSUB)],
                        agg_sh.at[pl.ds(s * _FSUB, _FSUB)])
        plsc.subcore_barrier()
        base = w * _FPW
        pltpu.sync_copy(dst_hbm.at[pl.ds(base, _FPW)], idx_v)
        pltpu.sync_copy(m_hbm.at[pl.ds(base, _FPW)], m_v)
        for j0 in range(0, _FPW, 16):
            for j in range(16):
                pltpu.async_copy(m_v.at[j0 + j], agg_sh.at[idx_v.at[j0 + j]],
                                 sem, add=True)
            pltpu.make_async_copy(
                m_hbm.at[pl.ds(0, 16)],
                m_v.at[pl.ds(j0, 16)], sem).wait()
        plsc.subcore_barrier()
        pltpu.sync_copy(agg_sh.at[pl.ds(s * _FSUB, _FSUB)],
                        out_hbm.at[c].at[pl.ds(s * _FSUB, _FSUB)])

    out = sk(mf, dst4, zeros)
    return out.reshape(2, N_PAD, 4)


def _in_layer(lp, h8, hpair, e, dst4, alpha):
    W1, b1 = lp["edge"][0]
    pieces = [(0, 0, 5), (8, 5, 5), (16, 10, 4)]
    elayers = _prep_layers(
        [(W1, b1)] + lp["edge"][1:], first_pieces=pieces, first_total=20
    )
    (m,) = _tc_mlp(
        [hpair, e], [{"layers": elayers, "out_act": None}], tile=EDGE_TILE
    )
    agg2 = _sc_scatter(m, dst4)
    Wn, bn = lp["node"][0]
    npieces = [(0, 0, 5), (8, 5, 4)]
    nlayers = _prep_layers(
        [(Wn, bn)] + lp["node"][1:], first_pieces=npieces, first_total=12,
        out_pad=8,
    )
    agg = agg2[0] + agg2[1]
    (h8_new,) = _tc_mlp(
        [h8, agg],
        [{"layers": nlayers, "out_act": None, "alpha": alpha}],
        tile=NODE_TILE,
    )
    return h8_new, m


def kernel(x, edge_index, edge_attr, params, alpha_ec=0.5, alpha_hc=0.5):
    src = edge_index[0]
    dst = edge_index[1]
    pad_idx = jnp.full((E_PAD - E_EDGES,), N_NODES, dtype=jnp.int32)
    src_p = jnp.concatenate([src, pad_idx])
    dst_p = jnp.concatenate([dst, pad_idx])
    ii3 = jnp.stack([dst_p, src_p], axis=1).reshape(_GCHUNKS, 128)
    dst4 = (dst_p[:, None] * 4
            + jnp.arange(4, dtype=jnp.int32)[None, :]).reshape(_FCHUNKS, 128)
    ea_p = _pad_rows(edge_attr, E_PAD)
    x_p = _pad_rows(x, N_PAD)

    enc_heads = []
    for name in ("ec_node_enc", "hc_node_enc"):
        enc_heads.append({
            "layers": _prep_layers(params[name], out_pad=8),
            "out_act": "relu",
        })
    h_ec8, h_hc8 = _tc_mlp([x_p], enc_heads, tile=NODE_TILE)

    (e_ec,) = _tc_mlp(
        [ea_p],
        [{"layers": _prep_layers(params["ec_edge_enc"]), "out_act": "relu"}],
        tile=EDGE_TILE,
    )

    e_list = [e_ec]
    e_cur = e_ec
    for lp in params["ec_layers"]:
        hpair = _sc_gather(h_ec8, ii3)
        h_ec8, m = _in_layer(lp, h_ec8, hpair, e_cur, dst4, alpha_ec)
        e_list.append(m)
        e_cur = m

    (ew_p,) = _tc_mlp(
        e_list,
        [{"layers": _prep_layers(params["W"]), "out_act": "sigmoid"}],
        tile=EDGE_TILE,
    )

    (e_hc,) = _tc_mlp(
        [ew_p, ea_p],
        [{"layers": _prep_layers(params["hc_edge_enc"]), "out_act": "relu"}],
        tile=EDGE_TILE,
    )

    eh_list = [e_hc]
    e_cur = e_hc
    for lp in params["hc_layers"]:
        hpair = _sc_gather(h_hc8, ii3)
        h_hc8, m = _in_layer(lp, h_hc8, hpair, e_cur, dst4, alpha_hc)
        eh_list.append(m)
        e_cur = m

    h5 = [(0, 0, 5)]
    beta_p, hout_p = _tc_mlp(
        [h_hc8],
        [
            {"layers": _prep_layers(params["B"], first_pieces=h5,
                                    first_total=8), "out_act": "sigmoid"},
            {"layers": _prep_layers(params["X"], first_pieces=h5,
                                    first_total=8), "out_act": None},
        ],
        tile=NODE_TILE,
    )

    Wp1, bp1 = params["P"]["edge"][0]
    ppieces = [(0, 0, 5), (8, 5, 5), (16, 10, 16)]
    playersE = _prep_layers(
        [(Wp1, bp1)] + params["P"]["edge"][1:],
        first_pieces=ppieces, first_total=32, out_pad=4,
    )
    hpair = _sc_gather(h_hc8, ii3)
    (mp,) = _tc_mlp(
        [hpair] + eh_list,
        [{"layers": playersE, "out_act": None}],
        tile=EDGE_TILE,
    )
    aggp2 = _sc_scatter(mp, dst4)
    aggp = aggp2[0] + aggp2[1]
    Wpn, bpn = params["P"]["node"][0]
    npieces = [(0, 0, 5), (8, 5, 1)]
    playersN = _prep_layers(
        [(Wpn, bpn)] + params["P"]["node"][1:],
        first_pieces=npieces, first_total=12,
    )
    (track_p,) = _tc_mlp(
        [h_hc8, aggp],
        [{"layers": playersN, "out_act": None}],
        tile=NODE_TILE,
    )

    edge_weights = ew_p[:E_EDGES]
    h = hout_p[:N_NODES]
    beta = beta_p[:N_NODES]
    track_params = track_p[:N_NODES]
    return (edge_weights, h, beta, track_params)
